# baseline (speedup 1.0000x reference)
import functools

import jax
import jax.numpy as jnp
from jax import lax
from jax.experimental import pallas as pl
from jax.experimental.pallas import tpu as pltpu
from jax.experimental.pallas import tpu_sc as plsc

NC = 2
NS = 16
EC = 80


CH = 128
NB = 2


def _sc_degree(dst_p, n_pad):
    ept = dst_p.shape[0] // (NC * NS)
    cpt = ept // CH
    assert cpt % NB == 0
    rpt = n_pad // NS
    mesh = plsc.VectorSubcoreMesh(core_axis_name="c", subcore_axis_name="s")

    @functools.partial(
        pl.kernel, mesh=mesh,
        out_type=jax.ShapeDtypeStruct((NC * n_pad,), jnp.float32),
        scratch_types=[
            [pltpu.VMEM((CH,), jnp.int32)] * NB,
            pltpu.VMEM((CH,), jnp.float32),
            pltpu.VMEM((rpt,), jnp.float32),
            pltpu.VMEM_SHARED((n_pad,), jnp.float32),
            [pltpu.SemaphoreType.DMA] * NB,
            [pltpu.SemaphoreType.DMA] * NB,
        ],
    )
    def k(dst_hbm, out_hbm, didx, ones_v, stage_v, acc_s, isem, ssem):
        c = lax.axis_index("c")
        s = lax.axis_index("s")
        base0 = (c * NS + s) * ept
        for j in range(CH // 16):
            ones_v[pl.ds(j * 16, 16)] = jnp.ones((16,), jnp.float32)

        def zloop(j, carry):
            stage_v[pl.ds(j * 16, 16)] = jnp.zeros((16,), jnp.float32)
            return carry

        lax.fori_loop(0, rpt // 16, zloop, 0)
        pltpu.sync_copy(stage_v, acc_s.at[pl.ds(s * rpt, rpt)])
        plsc.subcore_barrier()
        for b in range(NB - 1):
            pltpu.async_copy(dst_hbm.at[pl.ds(base0 + b * CH, CH)],
                             didx[b], isem[b])

        def visit(io, carry):
            for u in range(NB):
                j = io * NB + u
                bp = (u - 1) % NB
                pltpu.make_async_copy(dst_hbm.at[pl.ds(0, CH)],
                                      didx[u], isem[u]).wait()
                pltpu.async_copy(ones_v, acc_s.at[didx[u]], ssem[u], add=True)
                wait_prev = pltpu.make_async_copy(
                    out_hbm.at[pl.ds(0, CH)], ones_v, ssem[bp]).wait
                if u == 0:
                    pl.when(j >= 1)(wait_prev)
                else:
                    wait_prev()

                @pl.when(j + NB - 1 < cpt)
                def _():
                    pltpu.async_copy(
                        dst_hbm.at[pl.ds(base0 + (j + NB - 1) * CH, CH)],
                        didx[bp], isem[bp])
            return carry

        lax.fori_loop(0, cpt // NB, visit, 0)
        pltpu.make_async_copy(out_hbm.at[pl.ds(0, CH)], ones_v,
                              ssem[(cpt - 1) % NB]).wait()
        plsc.subcore_barrier()
        pltpu.sync_copy(acc_s.at[pl.ds(s * rpt, rpt)], stage_v)
        pltpu.sync_copy(stage_v, out_hbm.at[pl.ds(c * n_pad + s * rpt, rpt)])

    return k(dst_p)


_SC_SCATTER_CACHE = {}


def _sc_scatter(g, src_p, dst_p, zeros_nh):
    N, H = g.shape
    ept = src_p.shape[0] // (NC * NS)
    cpt = ept // CH
    assert cpt % NB == 0
    rpt = (N // (NS * 8)) * 8
    tail = N - NS * rpt
    mesh = plsc.VectorSubcoreMesh(core_axis_name="c", subcore_axis_name="s")

    key = (N, H, ept)
    if key in _SC_SCATTER_CACHE:
        return _SC_SCATTER_CACHE[key](g, src_p, dst_p, zeros_nh)

    @functools.partial(
        pl.kernel, mesh=mesh,
        out_type=jax.ShapeDtypeStruct((NC, N, H), jnp.float32),
        scratch_types=[
            pltpu.VMEM((ept,), jnp.int32),
            [pltpu.VMEM((CH,), jnp.int32)] * NB,
            [pltpu.VMEM((CH, H), jnp.float32)] * NB,
            pltpu.VMEM_SHARED((N + 64, H), jnp.float32),
            [pltpu.SemaphoreType.DMA] * NB,
            [pltpu.SemaphoreType.DMA] * NB,
            [pltpu.SemaphoreType.DMA] * NB,
        ],
    )
    def k(g_hbm, src_hbm, dst_hbm, zeros_hbm, out_hbm,
          sidx_v, didx, rows, acc_s, gsem, ssem, isem):
        c = lax.axis_index("c")
        s = lax.axis_index("s")
        base0 = (c * NS + s) * ept
        pltpu.sync_copy(src_hbm.at[pl.ds(base0, ept)], sidx_v)
        pltpu.sync_copy(zeros_hbm.at[pl.ds(s * rpt, rpt)],
                        acc_s.at[pl.ds(s * rpt, rpt)])

        @pl.when(s == NS - 1)
        def _():
            pltpu.sync_copy(zeros_hbm.at[pl.ds(NS * rpt, tail)],
                            acc_s.at[pl.ds(NS * rpt, tail)])

        plsc.subcore_barrier()

        def run(gref):
            for b in range(NB - 1):
                pltpu.async_copy(dst_hbm.at[pl.ds(base0 + b * CH, CH)],
                                 didx[b], isem[b])
                pltpu.async_copy(gref.at[sidx_v.at[pl.ds(b * CH, CH)]],
                                 rows[b], gsem[b])

            def visit(io, carry):
                for u in range(NB):
                    j = io * NB + u
                    bp = (u - 1) % NB
                    pltpu.make_async_copy(gref.at[pl.ds(0, CH)],
                                          rows[u], gsem[u]).wait()
                    pltpu.make_async_copy(dst_hbm.at[pl.ds(0, CH)],
                                          didx[u], isem[u]).wait()
                    pltpu.async_copy(rows[u], acc_s.at[didx[u]],
                                     ssem[u], add=True)
                    wait_prev = pltpu.make_async_copy(
                        gref.at[pl.ds(0, CH)], rows[bp], ssem[bp]).wait
                    if u == 0:
                        pl.when(j >= 1)(wait_prev)
                    else:
                        wait_prev()

                    @pl.when(j + NB - 1 < cpt)
                    def _():
                        pltpu.async_copy(
                            dst_hbm.at[pl.ds(base0 + (j + NB - 1) * CH, CH)],
                            didx[bp], isem[bp])
                        pltpu.async_copy(
                            gref.at[sidx_v.at[pl.ds((j + NB - 1) * CH, CH)]],
                            rows[bp], gsem[bp])
                return carry

            lax.fori_loop(0, cpt // NB, visit, 0)

        run(g_hbm)
        pltpu.make_async_copy(g_hbm.at[pl.ds(0, CH)],
                              rows[(cpt - 1) % NB], ssem[(cpt - 1) % NB]).wait()
        plsc.subcore_barrier()
        pltpu.sync_copy(acc_s.at[pl.ds(s * rpt, rpt)],
                        out_hbm.at[c, pl.ds(s * rpt, rpt)])

        @pl.when(s == NS - 1)
        def _():
            pltpu.sync_copy(acc_s.at[pl.ds(NS * rpt, tail)],
                            out_hbm.at[c, pl.ds(NS * rpt, tail)])

    _SC_SCATTER_CACHE[key] = k
    return k(g, src_p, dst_p, zeros_nh)


RB = 2000


def _tc_mm_scale(x, W, p0, p1):
    N, D = x.shape
    H = W.shape[1]
    nb = N // RB

    def body(x_ref, w_ref, p0_ref, p1_ref, g_ref, dis_ref):
        deg = p0_ref[...] + p1_ref[...] + 1.0
        dis = lax.rsqrt(jnp.maximum(deg, 1e-12))
        h = jnp.dot(x_ref[...], w_ref[...], preferred_element_type=jnp.float32)
        g_ref[...] = h * dis
        dis_ref[...] = dis

    return pl.pallas_call(
        body,
        grid=(nb,),
        in_specs=[
            pl.BlockSpec((RB, D), lambda i: (i, 0)),
            pl.BlockSpec((D, H), lambda i: (0, 0)),
            pl.BlockSpec((RB, 1), lambda i: (i, 0)),
            pl.BlockSpec((RB, 1), lambda i: (i, 0)),
        ],
        out_specs=[
            pl.BlockSpec((RB, H), lambda i: (i, 0)),
            pl.BlockSpec((RB, 1), lambda i: (i, 0)),
        ],
        out_shape=[
            jax.ShapeDtypeStruct((N, H), jnp.float32),
            jax.ShapeDtypeStruct((N, 1), jnp.float32),
        ],
    )(x, W, p0, p1)


def _tc_post(a0, a1, g, dis, b):
    N, H = g.shape
    nb = N // RB

    def body(a0_ref, a1_ref, g_ref, dis_ref, b_ref, t_ref, st_ref):
        i = pl.program_id(0)
        t = dis_ref[...] * (a0_ref[...] + a1_ref[...] + g_ref[...]) + b_ref[...][None, :]
        t_ref[...] = t

        @pl.when(i == 0)
        def _():
            st_ref[...] = jnp.zeros_like(st_ref)

        st_ref[0:1, :] += jnp.sum(t, axis=0, keepdims=True)
        st_ref[1:2, :] += jnp.sum(t * t, axis=0, keepdims=True)

    return pl.pallas_call(
        body,
        grid=(nb,),
        in_specs=[
            pl.BlockSpec((RB, H), lambda i: (i, 0)),
            pl.BlockSpec((RB, H), lambda i: (i, 0)),
            pl.BlockSpec((RB, H), lambda i: (i, 0)),
            pl.BlockSpec((RB, 1), lambda i: (i, 0)),
            pl.BlockSpec((H,), lambda i: (0,)),
        ],
        out_specs=[
            pl.BlockSpec((RB, H), lambda i: (i, 0)),
            pl.BlockSpec((2, H), lambda i: (0, 0)),
        ],
        out_shape=[
            jax.ShapeDtypeStruct((N, H), jnp.float32),
            jax.ShapeDtypeStruct((2, H), jnp.float32),
        ],
    )(a0, a1, g, dis, b)


def _tc_bn_mm(t, st, gamma, beta, dis, W):
    N, H = t.shape
    H2 = W.shape[1]
    nb = N // RB
    inv_n = 1.0 / N

    def body(t_ref, st_ref, ga_ref, be_ref, dis_ref, w_ref, g_ref):
        mu = st_ref[0:1, :] * inv_n
        var = st_ref[1:2, :] * inv_n - mu * mu
        hn = (t_ref[...] - mu) * lax.rsqrt(var + 1e-5) * ga_ref[...][None, :] \
            + be_ref[...][None, :]
        h = jnp.maximum(hn, 0.0)
        g_ref[...] = dis_ref[...] * jnp.dot(
            h, w_ref[...], preferred_element_type=jnp.float32)

    return pl.pallas_call(
        body,
        grid=(nb,),
        in_specs=[
            pl.BlockSpec((RB, H), lambda i: (i, 0)),
            pl.BlockSpec((2, H), lambda i: (0, 0)),
            pl.BlockSpec((H,), lambda i: (0,)),
            pl.BlockSpec((H,), lambda i: (0,)),
            pl.BlockSpec((RB, 1), lambda i: (i, 0)),
            pl.BlockSpec((H, H2), lambda i: (0, 0)),
        ],
        out_specs=pl.BlockSpec((RB, H2), lambda i: (i, 0)),
        out_shape=jax.ShapeDtypeStruct((N, H2), jnp.float32),
    )(t, st, gamma, beta, dis, W)


def _tc_final(t, st, gamma, beta, att_w, Wc, bc, batch_col, ng):
    N, H = t.shape
    nb = N // RB
    inv_n = 1.0 / N

    def body(t_ref, st_ref, ga_ref, be_ref, aw_ref, wc_ref, bc_ref, b_ref,
             att_ref, log_ref, pool_s):
        i = pl.program_id(0)
        mu = st_ref[0:1, :] * inv_n
        var = st_ref[1:2, :] * inv_n - mu * mu
        hn = (t_ref[...] - mu) * lax.rsqrt(var + 1e-5) * ga_ref[...][None, :] \
            + be_ref[...][None, :]
        h = jnp.maximum(hn, 0.0)
        att = jax.nn.sigmoid(
            jnp.dot(h, aw_ref[...], preferred_element_type=jnp.float32))
        att_ref[...] = att
        w = h * att
        oh = (lax.broadcasted_iota(jnp.int32, (RB, ng), 1)
              == b_ref[...]).astype(jnp.float32)
        part = lax.dot_general(oh, w, (((0,), (0,)), ((), ())),
                               preferred_element_type=jnp.float32,
                               precision=lax.Precision.HIGHEST)

        @pl.when(i == 0)
        def _():
            pool_s[...] = jnp.zeros_like(pool_s)

        pool_s[...] += part

        @pl.when(i == nb - 1)
        def _():
            log_ref[...] = jnp.dot(
                pool_s[...], wc_ref[...],
                preferred_element_type=jnp.float32) + bc_ref[...][None, :]

    return pl.pallas_call(
        body,
        grid=(nb,),
        in_specs=[
            pl.BlockSpec((RB, H), lambda i: (i, 0)),
            pl.BlockSpec((2, H), lambda i: (0, 0)),
            pl.BlockSpec((H,), lambda i: (0,)),
            pl.BlockSpec((H,), lambda i: (0,)),
            pl.BlockSpec((H, 1), lambda i: (0, 0)),
            pl.BlockSpec((H, 1), lambda i: (0, 0)),
            pl.BlockSpec((1,), lambda i: (0,)),
            pl.BlockSpec((RB, 1), lambda i: (i, 0)),
        ],
        out_specs=[
            pl.BlockSpec((RB, 1), lambda i: (i, 0)),
            pl.BlockSpec((ng, 1), lambda i: (0, 0)),
        ],
        out_shape=[
            jax.ShapeDtypeStruct((N, 1), jnp.float32),
            jax.ShapeDtypeStruct((ng, 1), jnp.float32),
        ],
        scratch_shapes=[pltpu.VMEM((ng, H), jnp.float32)],
    )(t, st, gamma, beta, att_w, Wc, bc, batch_col)


def kernel(x, edge_index, batch, W1, b1, W2, b2, bn1_gamma, bn1_beta,
           bn2_gamma, bn2_beta, att_w, Wc, bc):
    N, D = x.shape
    H = W1.shape[1]
    E = edge_index.shape[1]
    ng = 64
    batch_col = batch.reshape(N, 1)

    quantum = NC * NS * CH * NB
    e_tot = ((E + quantum - 1) // quantum) * quantum
    src_p = jnp.concatenate(
        [edge_index[0], jnp.zeros((e_tot - E,), jnp.int32)])
    dst_p = jnp.concatenate(
        [edge_index[1],
         N + (jnp.arange(e_tot - E, dtype=jnp.int32) % 64)])

    n_pad = ((N + NS * 16 - 1) // (NS * 16)) * (NS * 16)
    zeros_nh = jnp.zeros((N, H), jnp.float32)

    degp = _sc_degree(dst_p, n_pad)
    p0 = degp[:N].reshape(N, 1)
    p1 = degp[n_pad:n_pad + N].reshape(N, 1)

    g1, dis = _tc_mm_scale(x, W1, p0, p1)
    acc1 = _sc_scatter(g1, src_p, dst_p, zeros_nh)
    t1, st1 = _tc_post(acc1[0], acc1[1], g1, dis, b1)
    g2 = _tc_bn_mm(t1, st1, bn1_gamma, bn1_beta, dis, W2)
    acc2 = _sc_scatter(g2, src_p, dst_p, zeros_nh)
    t2, st2 = _tc_post(acc2[0], acc2[1], g2, dis, b2)
    att, logits = _tc_final(t2, st2, bn2_gamma, bn2_beta, att_w, Wc,
                            bc, batch_col, ng)
    return (logits, att)


# speedup vs baseline: 1.0591x; 1.0591x over previous
import functools

import jax
import jax.numpy as jnp
from jax import lax
from jax.experimental import pallas as pl
from jax.experimental.pallas import tpu as pltpu
from jax.experimental.pallas import tpu_sc as plsc

NC = 2
NS = 16
EC = 80


CH = 128
NB = 2


def _sc_degree(dst_p, n_pad):
    ept = dst_p.shape[0] // (NC * NS)
    cpt = ept // CH
    assert cpt % NB == 0
    rpt = n_pad // NS
    mesh = plsc.VectorSubcoreMesh(core_axis_name="c", subcore_axis_name="s")

    @functools.partial(
        pl.kernel, mesh=mesh,
        out_type=jax.ShapeDtypeStruct((NC * n_pad,), jnp.float32),
        scratch_types=[
            [pltpu.VMEM((CH,), jnp.int32)] * NB,
            pltpu.VMEM((CH,), jnp.float32),
            pltpu.VMEM((rpt,), jnp.float32),
            pltpu.VMEM_SHARED((n_pad,), jnp.float32),
            [pltpu.SemaphoreType.DMA] * NB,
            [pltpu.SemaphoreType.DMA] * NB,
        ],
    )
    def k(dst_hbm, out_hbm, didx, ones_v, stage_v, acc_s, isem, ssem):
        c = lax.axis_index("c")
        s = lax.axis_index("s")
        base0 = (c * NS + s) * ept
        for j in range(CH // 16):
            ones_v[pl.ds(j * 16, 16)] = jnp.ones((16,), jnp.float32)

        def zloop(j, carry):
            stage_v[pl.ds(j * 16, 16)] = jnp.zeros((16,), jnp.float32)
            return carry

        lax.fori_loop(0, rpt // 16, zloop, 0)
        pltpu.sync_copy(stage_v, acc_s.at[pl.ds(s * rpt, rpt)])
        plsc.subcore_barrier()
        for b in range(NB - 1):
            pltpu.async_copy(dst_hbm.at[pl.ds(base0 + b * CH, CH)],
                             didx[b], isem[b])

        def visit(io, carry):
            for u in range(NB):
                j = io * NB + u
                bp = (u - 1) % NB
                pltpu.make_async_copy(dst_hbm.at[pl.ds(0, CH)],
                                      didx[u], isem[u]).wait()
                pltpu.async_copy(ones_v, acc_s.at[didx[u]], ssem[u], add=True)
                wait_prev = pltpu.make_async_copy(
                    out_hbm.at[pl.ds(0, CH)], ones_v, ssem[bp]).wait
                if u == 0:
                    pl.when(j >= 1)(wait_prev)
                else:
                    wait_prev()

                @pl.when(j + NB - 1 < cpt)
                def _():
                    pltpu.async_copy(
                        dst_hbm.at[pl.ds(base0 + (j + NB - 1) * CH, CH)],
                        didx[bp], isem[bp])
            return carry

        lax.fori_loop(0, cpt // NB, visit, 0)
        pltpu.make_async_copy(out_hbm.at[pl.ds(0, CH)], ones_v,
                              ssem[(cpt - 1) % NB]).wait()
        plsc.subcore_barrier()
        pltpu.sync_copy(acc_s.at[pl.ds(s * rpt, rpt)], stage_v)
        pltpu.sync_copy(stage_v, out_hbm.at[pl.ds(c * n_pad + s * rpt, rpt)])

    return k(dst_p)


_SC_SCATTER_CACHE = {}


def _sc_scatter(g, src_p, dst_p, zeros_nh, ept_split):
    N, H = g.shape
    ept0, ept1 = ept_split
    cpt0, cpt1 = ept0 // CH, ept1 // CH
    assert cpt0 % NB == 0 and cpt1 % NB == 0
    e0 = NS * ept0
    rpt = (N // (NS * 8)) * 8
    tail = N - NS * rpt
    mesh = plsc.VectorSubcoreMesh(core_axis_name="c", subcore_axis_name="s")

    key = (N, H, ept0, ept1)
    if key in _SC_SCATTER_CACHE:
        return _SC_SCATTER_CACHE[key](g, src_p, dst_p, zeros_nh)

    @functools.partial(
        pl.kernel, mesh=mesh,
        out_type=jax.ShapeDtypeStruct((NC, N, H), jnp.float32),
        scratch_types=[
            pltpu.VMEM((max(ept0, ept1),), jnp.int32),
            [pltpu.VMEM((CH,), jnp.int32)] * NB,
            [pltpu.VMEM((CH, H), jnp.float32)] * NB,
            pltpu.VMEM_SHARED((N + 64, H), jnp.float32),
            [pltpu.SemaphoreType.DMA] * NB,
            [pltpu.SemaphoreType.DMA] * NB,
            [pltpu.SemaphoreType.DMA] * NB,
        ],
    )
    def k(g_hbm, src_hbm, dst_hbm, zeros_hbm, out_hbm,
          sidx_v, didx, rows, acc_s, gsem, ssem, isem):
        c = lax.axis_index("c")
        s = lax.axis_index("s")
        base0 = jnp.where(c == 0, s * ept0, e0 + s * ept1)
        pltpu.sync_copy(zeros_hbm.at[pl.ds(s * rpt, rpt)],
                        acc_s.at[pl.ds(s * rpt, rpt)])

        @pl.when(s == NS - 1)
        def _():
            pltpu.sync_copy(zeros_hbm.at[pl.ds(NS * rpt, tail)],
                            acc_s.at[pl.ds(NS * rpt, tail)])

        plsc.subcore_barrier()

        def run(gref, ept, cpt):
            pltpu.sync_copy(src_hbm.at[pl.ds(base0, ept)],
                            sidx_v.at[pl.ds(0, ept)])
            for b in range(NB - 1):
                pltpu.async_copy(dst_hbm.at[pl.ds(base0 + b * CH, CH)],
                                 didx[b], isem[b])
                pltpu.async_copy(gref.at[sidx_v.at[pl.ds(b * CH, CH)]],
                                 rows[b], gsem[b])

            def visit(io, carry):
                for u in range(NB):
                    j = io * NB + u
                    bp = (u - 1) % NB
                    pltpu.make_async_copy(gref.at[pl.ds(0, CH)],
                                          rows[u], gsem[u]).wait()
                    pltpu.make_async_copy(dst_hbm.at[pl.ds(0, CH)],
                                          didx[u], isem[u]).wait()
                    pltpu.async_copy(rows[u], acc_s.at[didx[u]],
                                     ssem[u], add=True)
                    wait_prev = pltpu.make_async_copy(
                        gref.at[pl.ds(0, CH)], rows[bp], ssem[bp]).wait
                    if u == 0:
                        pl.when(j >= 1)(wait_prev)
                    else:
                        wait_prev()

                    @pl.when(j + NB - 1 < cpt)
                    def _():
                        pltpu.async_copy(
                            dst_hbm.at[pl.ds(base0 + (j + NB - 1) * CH, CH)],
                            didx[bp], isem[bp])
                        pltpu.async_copy(
                            gref.at[sidx_v.at[pl.ds((j + NB - 1) * CH, CH)]],
                            rows[bp], gsem[bp])
                return carry

            lax.fori_loop(0, cpt // NB, visit, 0)
            pltpu.make_async_copy(g_hbm.at[pl.ds(0, CH)],
                                  rows[(cpt - 1) % NB],
                                  ssem[(cpt - 1) % NB]).wait()

        @pl.when(c == 0)
        def _():
            run(g_hbm, ept0, cpt0)

        @pl.when(c == 1)
        def _():
            run(g_hbm, ept1, cpt1)
        plsc.subcore_barrier()
        pltpu.sync_copy(acc_s.at[pl.ds(s * rpt, rpt)],
                        out_hbm.at[c, pl.ds(s * rpt, rpt)])

        @pl.when(s == NS - 1)
        def _():
            pltpu.sync_copy(acc_s.at[pl.ds(NS * rpt, tail)],
                            out_hbm.at[c, pl.ds(NS * rpt, tail)])

    _SC_SCATTER_CACHE[key] = k
    return k(g, src_p, dst_p, zeros_nh)


RB = 2000


def _tc_mm_scale(x, W, p0, p1):
    N, D = x.shape
    H = W.shape[1]
    nb = N // RB

    def body(x_ref, w_ref, p0_ref, p1_ref, g_ref, dis_ref):
        deg = p0_ref[...] + p1_ref[...] + 1.0
        dis = lax.rsqrt(jnp.maximum(deg, 1e-12))
        h = jnp.dot(x_ref[...], w_ref[...], preferred_element_type=jnp.float32)
        g_ref[...] = h * dis
        dis_ref[...] = dis

    return pl.pallas_call(
        body,
        grid=(nb,),
        in_specs=[
            pl.BlockSpec((RB, D), lambda i: (i, 0)),
            pl.BlockSpec((D, H), lambda i: (0, 0)),
            pl.BlockSpec((RB, 1), lambda i: (i, 0)),
            pl.BlockSpec((RB, 1), lambda i: (i, 0)),
        ],
        out_specs=[
            pl.BlockSpec((RB, H), lambda i: (i, 0)),
            pl.BlockSpec((RB, 1), lambda i: (i, 0)),
        ],
        out_shape=[
            jax.ShapeDtypeStruct((N, H), jnp.float32),
            jax.ShapeDtypeStruct((N, 1), jnp.float32),
        ],
    )(x, W, p0, p1)


def _tc_post(a0, a1, g, dis, b):
    N, H = g.shape
    nb = N // RB

    def body(a0_ref, a1_ref, g_ref, dis_ref, b_ref, t_ref, st_ref):
        i = pl.program_id(0)
        t = dis_ref[...] * (a0_ref[...] + a1_ref[...] + g_ref[...]) + b_ref[...][None, :]
        t_ref[...] = t

        @pl.when(i == 0)
        def _():
            st_ref[...] = jnp.zeros_like(st_ref)

        st_ref[0:1, :] += jnp.sum(t, axis=0, keepdims=True)
        st_ref[1:2, :] += jnp.sum(t * t, axis=0, keepdims=True)

    return pl.pallas_call(
        body,
        grid=(nb,),
        in_specs=[
            pl.BlockSpec((RB, H), lambda i: (i, 0)),
            pl.BlockSpec((RB, H), lambda i: (i, 0)),
            pl.BlockSpec((RB, H), lambda i: (i, 0)),
            pl.BlockSpec((RB, 1), lambda i: (i, 0)),
            pl.BlockSpec((H,), lambda i: (0,)),
        ],
        out_specs=[
            pl.BlockSpec((RB, H), lambda i: (i, 0)),
            pl.BlockSpec((2, H), lambda i: (0, 0)),
        ],
        out_shape=[
            jax.ShapeDtypeStruct((N, H), jnp.float32),
            jax.ShapeDtypeStruct((2, H), jnp.float32),
        ],
    )(a0, a1, g, dis, b)


def _tc_bn_mm(t, st, gamma, beta, dis, W):
    N, H = t.shape
    H2 = W.shape[1]
    nb = N // RB
    inv_n = 1.0 / N

    def body(t_ref, st_ref, ga_ref, be_ref, dis_ref, w_ref, g_ref):
        mu = st_ref[0:1, :] * inv_n
        var = st_ref[1:2, :] * inv_n - mu * mu
        hn = (t_ref[...] - mu) * lax.rsqrt(var + 1e-5) * ga_ref[...][None, :] \
            + be_ref[...][None, :]
        h = jnp.maximum(hn, 0.0)
        g_ref[...] = dis_ref[...] * jnp.dot(
            h, w_ref[...], preferred_element_type=jnp.float32)

    return pl.pallas_call(
        body,
        grid=(nb,),
        in_specs=[
            pl.BlockSpec((RB, H), lambda i: (i, 0)),
            pl.BlockSpec((2, H), lambda i: (0, 0)),
            pl.BlockSpec((H,), lambda i: (0,)),
            pl.BlockSpec((H,), lambda i: (0,)),
            pl.BlockSpec((RB, 1), lambda i: (i, 0)),
            pl.BlockSpec((H, H2), lambda i: (0, 0)),
        ],
        out_specs=pl.BlockSpec((RB, H2), lambda i: (i, 0)),
        out_shape=jax.ShapeDtypeStruct((N, H2), jnp.float32),
    )(t, st, gamma, beta, dis, W)


def _tc_final(t, st, gamma, beta, att_w, Wc, bc, batch_col, ng):
    N, H = t.shape
    nb = N // RB
    inv_n = 1.0 / N

    def body(t_ref, st_ref, ga_ref, be_ref, aw_ref, wc_ref, bc_ref, b_ref,
             att_ref, log_ref, pool_s):
        i = pl.program_id(0)
        mu = st_ref[0:1, :] * inv_n
        var = st_ref[1:2, :] * inv_n - mu * mu
        hn = (t_ref[...] - mu) * lax.rsqrt(var + 1e-5) * ga_ref[...][None, :] \
            + be_ref[...][None, :]
        h = jnp.maximum(hn, 0.0)
        att = jax.nn.sigmoid(
            jnp.dot(h, aw_ref[...], preferred_element_type=jnp.float32))
        att_ref[...] = att
        w = h * att
        oh = (lax.broadcasted_iota(jnp.int32, (RB, ng), 1)
              == b_ref[...]).astype(jnp.float32)
        part = lax.dot_general(oh, w, (((0,), (0,)), ((), ())),
                               preferred_element_type=jnp.float32,
                               precision=lax.Precision.HIGHEST)

        @pl.when(i == 0)
        def _():
            pool_s[...] = jnp.zeros_like(pool_s)

        pool_s[...] += part

        @pl.when(i == nb - 1)
        def _():
            log_ref[...] = jnp.dot(
                pool_s[...], wc_ref[...],
                preferred_element_type=jnp.float32) + bc_ref[...][None, :]

    return pl.pallas_call(
        body,
        grid=(nb,),
        in_specs=[
            pl.BlockSpec((RB, H), lambda i: (i, 0)),
            pl.BlockSpec((2, H), lambda i: (0, 0)),
            pl.BlockSpec((H,), lambda i: (0,)),
            pl.BlockSpec((H,), lambda i: (0,)),
            pl.BlockSpec((H, 1), lambda i: (0, 0)),
            pl.BlockSpec((H, 1), lambda i: (0, 0)),
            pl.BlockSpec((1,), lambda i: (0,)),
            pl.BlockSpec((RB, 1), lambda i: (i, 0)),
        ],
        out_specs=[
            pl.BlockSpec((RB, 1), lambda i: (i, 0)),
            pl.BlockSpec((ng, 1), lambda i: (0, 0)),
        ],
        out_shape=[
            jax.ShapeDtypeStruct((N, 1), jnp.float32),
            jax.ShapeDtypeStruct((ng, 1), jnp.float32),
        ],
        scratch_shapes=[pltpu.VMEM((ng, H), jnp.float32)],
    )(t, st, gamma, beta, att_w, Wc, bc, batch_col)


def kernel(x, edge_index, batch, W1, b1, W2, b2, bn1_gamma, bn1_beta,
           bn2_gamma, bn2_beta, att_w, Wc, bc):
    N, D = x.shape
    H = W1.shape[1]
    E = edge_index.shape[1]
    ng = 64
    batch_col = batch.reshape(N, 1)

    unit = CH * NB * 2
    ept0 = ((E * 2 // 3) // (NS * unit)) * unit
    ept1 = ((E - NS * ept0 + NS * unit - 1) // (NS * unit)) * unit
    e_tot = NS * (ept0 + ept1)
    src_p = jnp.concatenate(
        [edge_index[0], jnp.zeros((e_tot - E,), jnp.int32)])
    dst_p = jnp.concatenate(
        [edge_index[1],
         N + (jnp.arange(e_tot - E, dtype=jnp.int32) % 64)])

    n_pad = ((N + NS * 16 - 1) // (NS * 16)) * (NS * 16)
    zeros_nh = jnp.zeros((N, H), jnp.float32)

    degp = _sc_degree(dst_p, n_pad)
    p0 = degp[:N].reshape(N, 1)
    p1 = degp[n_pad:n_pad + N].reshape(N, 1)

    g1, dis = _tc_mm_scale(x, W1, p0, p1)
    acc1 = _sc_scatter(g1, src_p, dst_p, zeros_nh, (ept0, ept1))
    t1, st1 = _tc_post(acc1[0], acc1[1], g1, dis, b1)
    g2 = _tc_bn_mm(t1, st1, bn1_gamma, bn1_beta, dis, W2)
    acc2 = _sc_scatter(g2, src_p, dst_p, zeros_nh, (ept0, ept1))
    t2, st2 = _tc_post(acc2[0], acc2[1], g2, dis, b2)
    att, logits = _tc_final(t2, st2, bn2_gamma, bn2_beta, att_w, Wc,
                            bc, batch_col, ng)
    return (logits, att)


# speedup vs baseline: 1.4480x; 1.3672x over previous
import functools

import jax
import jax.numpy as jnp
from jax import lax
from jax.experimental import pallas as pl
from jax.experimental.pallas import tpu as pltpu
from jax.experimental.pallas import tpu_sc as plsc

NC = 2
NS = 16
EC = 80


CH = 128
NB = 2


def _sc_degree(dst_p, n_pad):
    ept = dst_p.shape[0] // (NC * NS)
    cpt = ept // CH
    assert cpt % NB == 0
    rpt = n_pad // NS
    mesh = plsc.VectorSubcoreMesh(core_axis_name="c", subcore_axis_name="s")

    @functools.partial(
        pl.kernel, mesh=mesh,
        out_type=jax.ShapeDtypeStruct((NC * n_pad,), jnp.float32),
        scratch_types=[
            [pltpu.VMEM((CH,), jnp.int32)] * NB,
            pltpu.VMEM((CH,), jnp.float32),
            pltpu.VMEM((rpt,), jnp.float32),
            pltpu.VMEM_SHARED((n_pad,), jnp.float32),
            [pltpu.SemaphoreType.DMA] * NB,
            [pltpu.SemaphoreType.DMA] * NB,
        ],
    )
    def k(dst_hbm, out_hbm, didx, ones_v, stage_v, acc_s, isem, ssem):
        c = lax.axis_index("c")
        s = lax.axis_index("s")
        base0 = (c * NS + s) * ept
        for j in range(CH // 16):
            ones_v[pl.ds(j * 16, 16)] = jnp.ones((16,), jnp.float32)

        def zloop(j, carry):
            stage_v[pl.ds(j * 16, 16)] = jnp.zeros((16,), jnp.float32)
            return carry

        lax.fori_loop(0, rpt // 16, zloop, 0)
        pltpu.sync_copy(stage_v, acc_s.at[pl.ds(s * rpt, rpt)])
        plsc.subcore_barrier()
        for b in range(NB - 1):
            pltpu.async_copy(dst_hbm.at[pl.ds(base0 + b * CH, CH)],
                             didx[b], isem[b])

        def visit(io, carry):
            for u in range(NB):
                j = io * NB + u
                bp = (u - 1) % NB
                pltpu.make_async_copy(dst_hbm.at[pl.ds(0, CH)],
                                      didx[u], isem[u]).wait()
                pltpu.async_copy(ones_v, acc_s.at[didx[u]], ssem[u], add=True)
                wait_prev = pltpu.make_async_copy(
                    out_hbm.at[pl.ds(0, CH)], ones_v, ssem[bp]).wait
                if u == 0:
                    pl.when(j >= 1)(wait_prev)
                else:
                    wait_prev()

                @pl.when(j + NB - 1 < cpt)
                def _():
                    pltpu.async_copy(
                        dst_hbm.at[pl.ds(base0 + (j + NB - 1) * CH, CH)],
                        didx[bp], isem[bp])
            return carry

        lax.fori_loop(0, cpt // NB, visit, 0)
        pltpu.make_async_copy(out_hbm.at[pl.ds(0, CH)], ones_v,
                              ssem[(cpt - 1) % NB]).wait()
        plsc.subcore_barrier()
        pltpu.sync_copy(acc_s.at[pl.ds(s * rpt, rpt)], stage_v)
        pltpu.sync_copy(stage_v, out_hbm.at[pl.ds(c * n_pad + s * rpt, rpt)])

    return k(dst_p)


EC = 80


def _sc_scatter(g, src, dst, zeros_nh):
    N, H = g.shape
    E = src.shape[0]
    ept = E // (NC * NS)
    n_chunks = ept // EC
    rpt = (N // (NS * 8)) * 8
    tail = N - NS * rpt
    mesh = plsc.VectorSubcoreMesh(core_axis_name="c", subcore_axis_name="s")

    @functools.partial(
        pl.kernel, mesh=mesh,
        out_type=jax.ShapeDtypeStruct((NC, N, H), jnp.float32),
        scratch_types=[
            pltpu.VMEM((EC,), jnp.int32),
            pltpu.VMEM((EC,), jnp.int32),
            pltpu.VMEM((EC, H), jnp.float32),
            pltpu.VMEM_SHARED((N, H), jnp.float32),
            pltpu.SemaphoreType.DMA,
        ],
    )
    def k(g_hbm, src_hbm, dst_hbm, zeros_hbm, out_hbm,
          sidx_v, didx_v, rows_v, acc_s, sem):
        c = lax.axis_index("c")
        s = lax.axis_index("s")
        pltpu.sync_copy(zeros_hbm.at[pl.ds(s * rpt, rpt)],
                        acc_s.at[pl.ds(s * rpt, rpt)])

        @pl.when(s == NS - 1)
        def _():
            pltpu.sync_copy(zeros_hbm.at[pl.ds(NS * rpt, tail)],
                            acc_s.at[pl.ds(NS * rpt, tail)])

        plsc.subcore_barrier()
        base0 = (c * NS + s) * ept

        def chunk(i, carry):
            base = base0 + i * EC
            pltpu.sync_copy(src_hbm.at[pl.ds(base, EC)], sidx_v)
            pltpu.sync_copy(dst_hbm.at[pl.ds(base, EC)], didx_v)
            pltpu.async_copy(g_hbm.at[sidx_v], rows_v, sem).wait()
            pltpu.sync_copy(rows_v, acc_s.at[didx_v], add=True)
            return carry

        lax.fori_loop(0, n_chunks, chunk, 0)
        plsc.subcore_barrier()
        pltpu.sync_copy(acc_s.at[pl.ds(s * rpt, rpt)],
                        out_hbm.at[c, pl.ds(s * rpt, rpt)])

        @pl.when(s == NS - 1)
        def _():
            pltpu.sync_copy(acc_s.at[pl.ds(NS * rpt, tail)],
                            out_hbm.at[c, pl.ds(NS * rpt, tail)])

    return k(g, src, dst, zeros_nh)


RB = 2000


def _tc_mm_scale(x, W, p0, p1):
    N, D = x.shape
    H = W.shape[1]
    nb = N // RB

    def body(x_ref, w_ref, p0_ref, p1_ref, g_ref, dis_ref):
        deg = p0_ref[...] + p1_ref[...] + 1.0
        dis = lax.rsqrt(jnp.maximum(deg, 1e-12))
        h = jnp.dot(x_ref[...], w_ref[...], preferred_element_type=jnp.float32)
        g_ref[...] = h * dis
        dis_ref[...] = dis

    return pl.pallas_call(
        body,
        grid=(nb,),
        in_specs=[
            pl.BlockSpec((RB, D), lambda i: (i, 0)),
            pl.BlockSpec((D, H), lambda i: (0, 0)),
            pl.BlockSpec((RB, 1), lambda i: (i, 0)),
            pl.BlockSpec((RB, 1), lambda i: (i, 0)),
        ],
        out_specs=[
            pl.BlockSpec((RB, H), lambda i: (i, 0)),
            pl.BlockSpec((RB, 1), lambda i: (i, 0)),
        ],
        out_shape=[
            jax.ShapeDtypeStruct((N, H), jnp.float32),
            jax.ShapeDtypeStruct((N, 1), jnp.float32),
        ],
    )(x, W, p0, p1)


def _tc_post(a0, a1, g, dis, b):
    N, H = g.shape
    nb = N // RB

    def body(a0_ref, a1_ref, g_ref, dis_ref, b_ref, t_ref, st_ref):
        i = pl.program_id(0)
        t = dis_ref[...] * (a0_ref[...] + a1_ref[...] + g_ref[...]) + b_ref[...][None, :]
        t_ref[...] = t

        @pl.when(i == 0)
        def _():
            st_ref[...] = jnp.zeros_like(st_ref)

        st_ref[0:1, :] += jnp.sum(t, axis=0, keepdims=True)
        st_ref[1:2, :] += jnp.sum(t * t, axis=0, keepdims=True)

    return pl.pallas_call(
        body,
        grid=(nb,),
        in_specs=[
            pl.BlockSpec((RB, H), lambda i: (i, 0)),
            pl.BlockSpec((RB, H), lambda i: (i, 0)),
            pl.BlockSpec((RB, H), lambda i: (i, 0)),
            pl.BlockSpec((RB, 1), lambda i: (i, 0)),
            pl.BlockSpec((H,), lambda i: (0,)),
        ],
        out_specs=[
            pl.BlockSpec((RB, H), lambda i: (i, 0)),
            pl.BlockSpec((2, H), lambda i: (0, 0)),
        ],
        out_shape=[
            jax.ShapeDtypeStruct((N, H), jnp.float32),
            jax.ShapeDtypeStruct((2, H), jnp.float32),
        ],
    )(a0, a1, g, dis, b)


def _tc_bn_mm(t, st, gamma, beta, dis, W):
    N, H = t.shape
    H2 = W.shape[1]
    nb = N // RB
    inv_n = 1.0 / N

    def body(t_ref, st_ref, ga_ref, be_ref, dis_ref, w_ref, g_ref):
        mu = st_ref[0:1, :] * inv_n
        var = st_ref[1:2, :] * inv_n - mu * mu
        hn = (t_ref[...] - mu) * lax.rsqrt(var + 1e-5) * ga_ref[...][None, :] \
            + be_ref[...][None, :]
        h = jnp.maximum(hn, 0.0)
        g_ref[...] = dis_ref[...] * jnp.dot(
            h, w_ref[...], preferred_element_type=jnp.float32)

    return pl.pallas_call(
        body,
        grid=(nb,),
        in_specs=[
            pl.BlockSpec((RB, H), lambda i: (i, 0)),
            pl.BlockSpec((2, H), lambda i: (0, 0)),
            pl.BlockSpec((H,), lambda i: (0,)),
            pl.BlockSpec((H,), lambda i: (0,)),
            pl.BlockSpec((RB, 1), lambda i: (i, 0)),
            pl.BlockSpec((H, H2), lambda i: (0, 0)),
        ],
        out_specs=pl.BlockSpec((RB, H2), lambda i: (i, 0)),
        out_shape=jax.ShapeDtypeStruct((N, H2), jnp.float32),
    )(t, st, gamma, beta, dis, W)


def _tc_final(t, st, gamma, beta, att_w, Wc, bc, batch_col, ng):
    N, H = t.shape
    nb = N // RB
    inv_n = 1.0 / N

    def body(t_ref, st_ref, ga_ref, be_ref, aw_ref, wc_ref, bc_ref, b_ref,
             att_ref, log_ref, pool_s):
        i = pl.program_id(0)
        mu = st_ref[0:1, :] * inv_n
        var = st_ref[1:2, :] * inv_n - mu * mu
        hn = (t_ref[...] - mu) * lax.rsqrt(var + 1e-5) * ga_ref[...][None, :] \
            + be_ref[...][None, :]
        h = jnp.maximum(hn, 0.0)
        att = jax.nn.sigmoid(
            jnp.dot(h, aw_ref[...], preferred_element_type=jnp.float32))
        att_ref[...] = att
        w = h * att
        oh = (lax.broadcasted_iota(jnp.int32, (RB, ng), 1)
              == b_ref[...]).astype(jnp.float32)
        part = lax.dot_general(oh, w, (((0,), (0,)), ((), ())),
                               preferred_element_type=jnp.float32,
                               precision=lax.Precision.HIGHEST)

        @pl.when(i == 0)
        def _():
            pool_s[...] = jnp.zeros_like(pool_s)

        pool_s[...] += part

        @pl.when(i == nb - 1)
        def _():
            log_ref[...] = jnp.dot(
                pool_s[...], wc_ref[...],
                preferred_element_type=jnp.float32) + bc_ref[...][None, :]

    return pl.pallas_call(
        body,
        grid=(nb,),
        in_specs=[
            pl.BlockSpec((RB, H), lambda i: (i, 0)),
            pl.BlockSpec((2, H), lambda i: (0, 0)),
            pl.BlockSpec((H,), lambda i: (0,)),
            pl.BlockSpec((H,), lambda i: (0,)),
            pl.BlockSpec((H, 1), lambda i: (0, 0)),
            pl.BlockSpec((H, 1), lambda i: (0, 0)),
            pl.BlockSpec((1,), lambda i: (0,)),
            pl.BlockSpec((RB, 1), lambda i: (i, 0)),
        ],
        out_specs=[
            pl.BlockSpec((RB, 1), lambda i: (i, 0)),
            pl.BlockSpec((ng, 1), lambda i: (0, 0)),
        ],
        out_shape=[
            jax.ShapeDtypeStruct((N, 1), jnp.float32),
            jax.ShapeDtypeStruct((ng, 1), jnp.float32),
        ],
        scratch_shapes=[pltpu.VMEM((ng, H), jnp.float32)],
    )(t, st, gamma, beta, att_w, Wc, bc, batch_col)


def kernel(x, edge_index, batch, W1, b1, W2, b2, bn1_gamma, bn1_beta,
           bn2_gamma, bn2_beta, att_w, Wc, bc):
    N, D = x.shape
    H = W1.shape[1]
    E = edge_index.shape[1]
    ng = 64
    batch_col = batch.reshape(N, 1)

    unit = CH * NB * 2
    ept0 = ((E * 2 // 3) // (NS * unit)) * unit
    ept1 = ((E - NS * ept0 + NS * unit - 1) // (NS * unit)) * unit
    e_tot = NS * (ept0 + ept1)
    src_p = jnp.concatenate(
        [edge_index[0], jnp.zeros((e_tot - E,), jnp.int32)])
    dst_p = jnp.concatenate(
        [edge_index[1],
         N + (jnp.arange(e_tot - E, dtype=jnp.int32) % 64)])

    n_pad = ((N + NS * 16 - 1) // (NS * 16)) * (NS * 16)
    zeros_nh = jnp.zeros((N, H), jnp.float32)

    degp = _sc_degree(dst_p, n_pad)
    p0 = degp[:N].reshape(N, 1)
    p1 = degp[n_pad:n_pad + N].reshape(N, 1)

    g1, dis = _tc_mm_scale(x, W1, p0, p1)
    acc1 = _sc_scatter(g1, edge_index[0], edge_index[1], zeros_nh)
    t1, st1 = _tc_post(acc1[0], acc1[1], g1, dis, b1)
    g2 = _tc_bn_mm(t1, st1, bn1_gamma, bn1_beta, dis, W2)
    acc2 = _sc_scatter(g2, edge_index[0], edge_index[1], zeros_nh)
    t2, st2 = _tc_post(acc2[0], acc2[1], g2, dis, b2)
    att, logits = _tc_final(t2, st2, bn2_gamma, bn2_beta, att_w, Wc,
                            bc, batch_col, ng)
    return (logits, att)


# speedup vs baseline: 2.4734x; 1.7081x over previous
import functools

import jax
import jax.numpy as jnp
from jax import lax
from jax.experimental import pallas as pl
from jax.experimental.pallas import tpu as pltpu
from jax.experimental.pallas import tpu_sc as plsc

NC = 2
NS = 16
EC = 80


CH = 128
NB = 2


def _sc_degree(dst_p, n_pad):
    ept = dst_p.shape[0] // (NC * NS)
    cpt = ept // CH
    assert cpt % NB == 0
    rpt = n_pad // NS
    mesh = plsc.VectorSubcoreMesh(core_axis_name="c", subcore_axis_name="s")

    @functools.partial(
        pl.kernel, mesh=mesh,
        out_type=jax.ShapeDtypeStruct((NC * n_pad,), jnp.float32),
        scratch_types=[
            [pltpu.VMEM((CH,), jnp.int32)] * NB,
            pltpu.VMEM((CH,), jnp.float32),
            pltpu.VMEM((rpt,), jnp.float32),
            pltpu.VMEM_SHARED((n_pad,), jnp.float32),
            [pltpu.SemaphoreType.DMA] * NB,
            [pltpu.SemaphoreType.DMA] * NB,
        ],
    )
    def k(dst_hbm, out_hbm, didx, ones_v, stage_v, acc_s, isem, ssem):
        c = lax.axis_index("c")
        s = lax.axis_index("s")
        base0 = (c * NS + s) * ept
        for j in range(CH // 16):
            ones_v[pl.ds(j * 16, 16)] = jnp.ones((16,), jnp.float32)

        def zloop(j, carry):
            stage_v[pl.ds(j * 16, 16)] = jnp.zeros((16,), jnp.float32)
            return carry

        lax.fori_loop(0, rpt // 16, zloop, 0)
        pltpu.sync_copy(stage_v, acc_s.at[pl.ds(s * rpt, rpt)])
        plsc.subcore_barrier()
        for b in range(NB - 1):
            pltpu.async_copy(dst_hbm.at[pl.ds(base0 + b * CH, CH)],
                             didx[b], isem[b])

        def visit(io, carry):
            for u in range(NB):
                j = io * NB + u
                bp = (u - 1) % NB
                pltpu.make_async_copy(dst_hbm.at[pl.ds(0, CH)],
                                      didx[u], isem[u]).wait()
                pltpu.async_copy(ones_v, acc_s.at[didx[u]], ssem[u], add=True)
                wait_prev = pltpu.make_async_copy(
                    out_hbm.at[pl.ds(0, CH)], ones_v, ssem[bp]).wait
                if u == 0:
                    pl.when(j >= 1)(wait_prev)
                else:
                    wait_prev()

                @pl.when(j + NB - 1 < cpt)
                def _():
                    pltpu.async_copy(
                        dst_hbm.at[pl.ds(base0 + (j + NB - 1) * CH, CH)],
                        didx[bp], isem[bp])
            return carry

        lax.fori_loop(0, cpt // NB, visit, 0)
        pltpu.make_async_copy(out_hbm.at[pl.ds(0, CH)], ones_v,
                              ssem[(cpt - 1) % NB]).wait()
        plsc.subcore_barrier()
        pltpu.sync_copy(acc_s.at[pl.ds(s * rpt, rpt)], stage_v)
        pltpu.sync_copy(stage_v, out_hbm.at[pl.ds(c * n_pad + s * rpt, rpt)])

    return k(dst_p)


EC = 80


def _sc_scatter(g, e3, zeros_nh):
    N, H = g.shape
    nct = e3.shape[0]
    n_chunks = nct // (NC * NS)
    half = (n_chunks - 1) // 2
    rpt = (N // (NS * 8)) * 8
    tail = N - NS * rpt
    mesh = plsc.VectorSubcoreMesh(core_axis_name="c", subcore_axis_name="s")

    @functools.partial(
        pl.kernel, mesh=mesh,
        out_type=jax.ShapeDtypeStruct((NC, N, H), jnp.float32),
        scratch_types=[
            [pltpu.VMEM((2, EC), jnp.int32)] * 2,
            [pltpu.VMEM((EC, H), jnp.float32)] * 2,
            pltpu.VMEM_SHARED((N, H), jnp.float32),
            [pltpu.SemaphoreType.DMA] * 2,
        ],
    )
    def k(g_hbm, e3_hbm, zeros_hbm, out_hbm, eidx, rows, acc_s, gsem):
        c = lax.axis_index("c")
        s = lax.axis_index("s")
        pltpu.sync_copy(zeros_hbm.at[pl.ds(s * rpt, rpt)],
                        acc_s.at[pl.ds(s * rpt, rpt)])

        @pl.when(s == NS - 1)
        def _():
            pltpu.sync_copy(zeros_hbm.at[pl.ds(NS * rpt, tail)],
                            acc_s.at[pl.ds(NS * rpt, tail)])

        plsc.subcore_barrier()
        gbase = (c * NS + s) * n_chunks
        pltpu.sync_copy(e3_hbm.at[gbase], eidx[0])
        pltpu.async_copy(g_hbm.at[eidx[0].at[0]], rows[0], gsem[0])

        def visit(io, carry):
            for u in range(2):
                j = io * 2 + u
                bp = 1 - u
                pltpu.sync_copy(e3_hbm.at[gbase + j + 1], eidx[bp])
                pltpu.async_copy(g_hbm.at[eidx[bp].at[0]], rows[bp], gsem[bp])
                pltpu.make_async_copy(g_hbm.at[pl.ds(0, EC)],
                                      rows[u], gsem[u]).wait()
                pltpu.sync_copy(rows[u], acc_s.at[eidx[u].at[1]], add=True)
            return carry

        lax.fori_loop(0, half, visit, 0)

        def fin(j, carry):
            u = 0
            pltpu.make_async_copy(g_hbm.at[pl.ds(0, EC)],
                                  rows[u], gsem[u]).wait()
            pltpu.sync_copy(rows[u], acc_s.at[eidx[u].at[1]], add=True)

            @pl.when(j + 1 < n_chunks)
            def _():
                pltpu.sync_copy(e3_hbm.at[gbase + j + 1], eidx[u])
                pltpu.async_copy(g_hbm.at[eidx[u].at[0]], rows[u], gsem[u])
            return carry

        lax.fori_loop(2 * half, n_chunks, fin, 0)
        plsc.subcore_barrier()
        pltpu.sync_copy(acc_s.at[pl.ds(s * rpt, rpt)],
                        out_hbm.at[c, pl.ds(s * rpt, rpt)])

        @pl.when(s == NS - 1)
        def _():
            pltpu.sync_copy(acc_s.at[pl.ds(NS * rpt, tail)],
                            out_hbm.at[c, pl.ds(NS * rpt, tail)])

    return k(g, e3, zeros_nh)


RB = 2000


def _tc_mm_scale(x, W, p0, p1):
    N, D = x.shape
    H = W.shape[1]
    nb = N // RB

    def body(x_ref, w_ref, p0_ref, p1_ref, g_ref, dis_ref):
        deg = p0_ref[...] + p1_ref[...] + 1.0
        dis = lax.rsqrt(jnp.maximum(deg, 1e-12))
        h = jnp.dot(x_ref[...], w_ref[...], preferred_element_type=jnp.float32)
        g_ref[...] = h * dis
        dis_ref[...] = dis

    return pl.pallas_call(
        body,
        grid=(nb,),
        in_specs=[
            pl.BlockSpec((RB, D), lambda i: (i, 0)),
            pl.BlockSpec((D, H), lambda i: (0, 0)),
            pl.BlockSpec((RB, 1), lambda i: (i, 0)),
            pl.BlockSpec((RB, 1), lambda i: (i, 0)),
        ],
        out_specs=[
            pl.BlockSpec((RB, H), lambda i: (i, 0)),
            pl.BlockSpec((RB, 1), lambda i: (i, 0)),
        ],
        out_shape=[
            jax.ShapeDtypeStruct((N, H), jnp.float32),
            jax.ShapeDtypeStruct((N, 1), jnp.float32),
        ],
    )(x, W, p0, p1)


def _tc_post(a0, a1, g, dis, b):
    N, H = g.shape
    nb = N // RB

    def body(a0_ref, a1_ref, g_ref, dis_ref, b_ref, t_ref, st_ref):
        i = pl.program_id(0)
        t = dis_ref[...] * (a0_ref[...] + a1_ref[...] + g_ref[...]) + b_ref[...][None, :]
        t_ref[...] = t

        @pl.when(i == 0)
        def _():
            st_ref[...] = jnp.zeros_like(st_ref)

        st_ref[0:1, :] += jnp.sum(t, axis=0, keepdims=True)
        st_ref[1:2, :] += jnp.sum(t * t, axis=0, keepdims=True)

    return pl.pallas_call(
        body,
        grid=(nb,),
        in_specs=[
            pl.BlockSpec((RB, H), lambda i: (i, 0)),
            pl.BlockSpec((RB, H), lambda i: (i, 0)),
            pl.BlockSpec((RB, H), lambda i: (i, 0)),
            pl.BlockSpec((RB, 1), lambda i: (i, 0)),
            pl.BlockSpec((H,), lambda i: (0,)),
        ],
        out_specs=[
            pl.BlockSpec((RB, H), lambda i: (i, 0)),
            pl.BlockSpec((2, H), lambda i: (0, 0)),
        ],
        out_shape=[
            jax.ShapeDtypeStruct((N, H), jnp.float32),
            jax.ShapeDtypeStruct((2, H), jnp.float32),
        ],
    )(a0, a1, g, dis, b)


def _tc_bn_mm(t, st, gamma, beta, dis, W):
    N, H = t.shape
    H2 = W.shape[1]
    nb = N // RB
    inv_n = 1.0 / N

    def body(t_ref, st_ref, ga_ref, be_ref, dis_ref, w_ref, g_ref):
        mu = st_ref[0:1, :] * inv_n
        var = st_ref[1:2, :] * inv_n - mu * mu
        hn = (t_ref[...] - mu) * lax.rsqrt(var + 1e-5) * ga_ref[...][None, :] \
            + be_ref[...][None, :]
        h = jnp.maximum(hn, 0.0)
        g_ref[...] = dis_ref[...] * jnp.dot(
            h, w_ref[...], preferred_element_type=jnp.float32)

    return pl.pallas_call(
        body,
        grid=(nb,),
        in_specs=[
            pl.BlockSpec((RB, H), lambda i: (i, 0)),
            pl.BlockSpec((2, H), lambda i: (0, 0)),
            pl.BlockSpec((H,), lambda i: (0,)),
            pl.BlockSpec((H,), lambda i: (0,)),
            pl.BlockSpec((RB, 1), lambda i: (i, 0)),
            pl.BlockSpec((H, H2), lambda i: (0, 0)),
        ],
        out_specs=pl.BlockSpec((RB, H2), lambda i: (i, 0)),
        out_shape=jax.ShapeDtypeStruct((N, H2), jnp.float32),
    )(t, st, gamma, beta, dis, W)


def _tc_final(t, st, gamma, beta, att_w, Wc, bc, batch_col, ng):
    N, H = t.shape
    nb = N // RB
    inv_n = 1.0 / N

    def body(t_ref, st_ref, ga_ref, be_ref, aw_ref, wc_ref, bc_ref, b_ref,
             att_ref, log_ref, pool_s):
        i = pl.program_id(0)
        mu = st_ref[0:1, :] * inv_n
        var = st_ref[1:2, :] * inv_n - mu * mu
        hn = (t_ref[...] - mu) * lax.rsqrt(var + 1e-5) * ga_ref[...][None, :] \
            + be_ref[...][None, :]
        h = jnp.maximum(hn, 0.0)
        att = jax.nn.sigmoid(
            jnp.dot(h, aw_ref[...], preferred_element_type=jnp.float32))
        att_ref[...] = att
        w = h * att
        oh = (lax.broadcasted_iota(jnp.int32, (RB, ng), 1)
              == b_ref[...]).astype(jnp.float32)
        part = lax.dot_general(oh, w, (((0,), (0,)), ((), ())),
                               preferred_element_type=jnp.float32,
                               precision=lax.Precision.HIGHEST)

        @pl.when(i == 0)
        def _():
            pool_s[...] = jnp.zeros_like(pool_s)

        pool_s[...] += part

        @pl.when(i == nb - 1)
        def _():
            log_ref[...] = jnp.dot(
                pool_s[...], wc_ref[...],
                preferred_element_type=jnp.float32) + bc_ref[...][None, :]

    return pl.pallas_call(
        body,
        grid=(nb,),
        in_specs=[
            pl.BlockSpec((RB, H), lambda i: (i, 0)),
            pl.BlockSpec((2, H), lambda i: (0, 0)),
            pl.BlockSpec((H,), lambda i: (0,)),
            pl.BlockSpec((H,), lambda i: (0,)),
            pl.BlockSpec((H, 1), lambda i: (0, 0)),
            pl.BlockSpec((H, 1), lambda i: (0, 0)),
            pl.BlockSpec((1,), lambda i: (0,)),
            pl.BlockSpec((RB, 1), lambda i: (i, 0)),
        ],
        out_specs=[
            pl.BlockSpec((RB, 1), lambda i: (i, 0)),
            pl.BlockSpec((ng, 1), lambda i: (0, 0)),
        ],
        out_shape=[
            jax.ShapeDtypeStruct((N, 1), jnp.float32),
            jax.ShapeDtypeStruct((ng, 1), jnp.float32),
        ],
        scratch_shapes=[pltpu.VMEM((ng, H), jnp.float32)],
    )(t, st, gamma, beta, att_w, Wc, bc, batch_col)


def kernel(x, edge_index, batch, W1, b1, W2, b2, bn1_gamma, bn1_beta,
           bn2_gamma, bn2_beta, att_w, Wc, bc):
    N, D = x.shape
    H = W1.shape[1]
    E = edge_index.shape[1]
    ng = 64
    batch_col = batch.reshape(N, 1)

    unit = CH * NB * 2
    ept0 = ((E * 2 // 3) // (NS * unit)) * unit
    ept1 = ((E - NS * ept0 + NS * unit - 1) // (NS * unit)) * unit
    e_tot = NS * (ept0 + ept1)
    src_p = jnp.concatenate(
        [edge_index[0], jnp.zeros((e_tot - E,), jnp.int32)])
    dst_p = jnp.concatenate(
        [edge_index[1],
         N + (jnp.arange(e_tot - E, dtype=jnp.int32) % 64)])

    n_pad = ((N + NS * 16 - 1) // (NS * 16)) * (NS * 16)
    zeros_nh = jnp.zeros((N, H), jnp.float32)

    e3 = jnp.stack([edge_index[0].reshape(-1, EC),
                    edge_index[1].reshape(-1, EC)], axis=1)
    degp = _sc_degree(dst_p, n_pad)
    p0 = degp[:N].reshape(N, 1)
    p1 = degp[n_pad:n_pad + N].reshape(N, 1)

    g1, dis = _tc_mm_scale(x, W1, p0, p1)
    acc1 = _sc_scatter(g1, e3, zeros_nh)
    t1, st1 = _tc_post(acc1[0], acc1[1], g1, dis, b1)
    g2 = _tc_bn_mm(t1, st1, bn1_gamma, bn1_beta, dis, W2)
    acc2 = _sc_scatter(g2, e3, zeros_nh)
    t2, st2 = _tc_post(acc2[0], acc2[1], g2, dis, b2)
    att, logits = _tc_final(t2, st2, bn2_gamma, bn2_beta, att_w, Wc,
                            bc, batch_col, ng)
    return (logits, att)


# speedup vs baseline: 2.4735x; 1.0000x over previous
import functools

import jax
import jax.numpy as jnp
from jax import lax
from jax.experimental import pallas as pl
from jax.experimental.pallas import tpu as pltpu
from jax.experimental.pallas import tpu_sc as plsc

NC = 2
NS = 16
EC = 80


CH = 128
NB = 2


def _sc_degree(dst_p, n_pad):
    ept = dst_p.shape[0] // (NC * NS)
    cpt = ept // CH
    assert cpt % NB == 0
    rpt = n_pad // NS
    mesh = plsc.VectorSubcoreMesh(core_axis_name="c", subcore_axis_name="s")

    @functools.partial(
        pl.kernel, mesh=mesh,
        out_type=jax.ShapeDtypeStruct((NC * n_pad,), jnp.float32),
        scratch_types=[
            [pltpu.VMEM((CH,), jnp.int32)] * NB,
            pltpu.VMEM((CH,), jnp.float32),
            pltpu.VMEM((rpt,), jnp.float32),
            pltpu.VMEM_SHARED((n_pad,), jnp.float32),
            [pltpu.SemaphoreType.DMA] * NB,
            [pltpu.SemaphoreType.DMA] * NB,
        ],
    )
    def k(dst_hbm, out_hbm, didx, ones_v, stage_v, acc_s, isem, ssem):
        c = lax.axis_index("c")
        s = lax.axis_index("s")
        base0 = (c * NS + s) * ept
        for j in range(CH // 16):
            ones_v[pl.ds(j * 16, 16)] = jnp.ones((16,), jnp.float32)

        def zloop(j, carry):
            stage_v[pl.ds(j * 16, 16)] = jnp.zeros((16,), jnp.float32)
            return carry

        lax.fori_loop(0, rpt // 16, zloop, 0)
        pltpu.sync_copy(stage_v, acc_s.at[pl.ds(s * rpt, rpt)])
        plsc.subcore_barrier()
        for b in range(NB - 1):
            pltpu.async_copy(dst_hbm.at[pl.ds(base0 + b * CH, CH)],
                             didx[b], isem[b])

        def visit(io, carry):
            for u in range(NB):
                j = io * NB + u
                bp = (u - 1) % NB
                pltpu.make_async_copy(dst_hbm.at[pl.ds(0, CH)],
                                      didx[u], isem[u]).wait()
                pltpu.async_copy(ones_v, acc_s.at[didx[u]], ssem[u], add=True)
                wait_prev = pltpu.make_async_copy(
                    out_hbm.at[pl.ds(0, CH)], ones_v, ssem[bp]).wait
                if u == 0:
                    pl.when(j >= 1)(wait_prev)
                else:
                    wait_prev()

                @pl.when(j + NB - 1 < cpt)
                def _():
                    pltpu.async_copy(
                        dst_hbm.at[pl.ds(base0 + (j + NB - 1) * CH, CH)],
                        didx[bp], isem[bp])
            return carry

        lax.fori_loop(0, cpt // NB, visit, 0)
        pltpu.make_async_copy(out_hbm.at[pl.ds(0, CH)], ones_v,
                              ssem[(cpt - 1) % NB]).wait()
        plsc.subcore_barrier()
        pltpu.sync_copy(acc_s.at[pl.ds(s * rpt, rpt)], stage_v)
        pltpu.sync_copy(stage_v, out_hbm.at[pl.ds(c * n_pad + s * rpt, rpt)])

    return k(dst_p)


EC = 80


def _sc_scatter(g, e3, zeros_nh):
    N, H = g.shape
    nct = e3.shape[0]
    n_chunks = nct // (NC * NS)
    half = (n_chunks - 1) // 2
    rpt = (N // (NS * 8)) * 8
    tail = N - NS * rpt
    mesh = plsc.VectorSubcoreMesh(core_axis_name="c", subcore_axis_name="s")

    @functools.partial(
        pl.kernel, mesh=mesh,
        out_type=jax.ShapeDtypeStruct((NC, N, H), jnp.float32),
        scratch_types=[
            [pltpu.VMEM((2, EC), jnp.int32)] * 2,
            [pltpu.VMEM((EC, H), jnp.float32)] * 2,
            pltpu.VMEM_SHARED((N, H), jnp.float32),
            [pltpu.SemaphoreType.DMA] * 2,
        ],
    )
    def k(g_hbm, e3_hbm, zeros_hbm, out_hbm, eidx, rows, acc_s, gsem):
        c = lax.axis_index("c")
        s = lax.axis_index("s")
        pltpu.sync_copy(zeros_hbm.at[pl.ds(s * rpt, rpt)],
                        acc_s.at[pl.ds(s * rpt, rpt)])

        @pl.when(s == NS - 1)
        def _():
            pltpu.sync_copy(zeros_hbm.at[pl.ds(NS * rpt, tail)],
                            acc_s.at[pl.ds(NS * rpt, tail)])

        plsc.subcore_barrier()
        gbase = (c * NS + s) * n_chunks
        pltpu.sync_copy(e3_hbm.at[gbase], eidx[0])
        pltpu.async_copy(g_hbm.at[eidx[0].at[0]], rows[0], gsem[0])

        def visit(io, carry):
            for u in range(2):
                j = io * 2 + u
                bp = 1 - u
                pltpu.sync_copy(e3_hbm.at[gbase + j + 1], eidx[bp])
                pltpu.async_copy(g_hbm.at[eidx[bp].at[0]], rows[bp], gsem[bp])
                pltpu.make_async_copy(g_hbm.at[pl.ds(0, EC)],
                                      rows[u], gsem[u]).wait()
                pltpu.sync_copy(rows[u], acc_s.at[eidx[u].at[1]], add=True)
            return carry

        lax.fori_loop(0, half, visit, 0)

        def fin(j, carry):
            u = 0
            pltpu.make_async_copy(g_hbm.at[pl.ds(0, EC)],
                                  rows[u], gsem[u]).wait()
            pltpu.sync_copy(rows[u], acc_s.at[eidx[u].at[1]], add=True)

            @pl.when(j + 1 < n_chunks)
            def _():
                pltpu.sync_copy(e3_hbm.at[gbase + j + 1], eidx[u])
                pltpu.async_copy(g_hbm.at[eidx[u].at[0]], rows[u], gsem[u])
            return carry

        lax.fori_loop(2 * half, n_chunks, fin, 0)
        plsc.subcore_barrier()
        pltpu.sync_copy(acc_s.at[pl.ds(s * rpt, rpt)],
                        out_hbm.at[c, pl.ds(s * rpt, rpt)])

        @pl.when(s == NS - 1)
        def _():
            pltpu.sync_copy(acc_s.at[pl.ds(NS * rpt, tail)],
                            out_hbm.at[c, pl.ds(NS * rpt, tail)])

    return k(g, e3, zeros_nh)


RB = 2000


def _tc_mm_scale(x, W, p0, p1):
    N, D = x.shape
    H = W.shape[1]
    nb = N // RB

    def body(x_ref, w_ref, p0_ref, p1_ref, g_ref, dis_ref):
        deg = p0_ref[...] + p1_ref[...] + 1.0
        dis = lax.rsqrt(jnp.maximum(deg, 1e-12))
        h = jnp.dot(x_ref[...], w_ref[...], preferred_element_type=jnp.float32)
        g_ref[...] = h * dis
        dis_ref[...] = dis

    return pl.pallas_call(
        body,
        grid=(nb,),
        in_specs=[
            pl.BlockSpec((RB, D), lambda i: (i, 0)),
            pl.BlockSpec((D, H), lambda i: (0, 0)),
            pl.BlockSpec((RB, 1), lambda i: (i, 0)),
            pl.BlockSpec((RB, 1), lambda i: (i, 0)),
        ],
        out_specs=[
            pl.BlockSpec((RB, H), lambda i: (i, 0)),
            pl.BlockSpec((RB, 1), lambda i: (i, 0)),
        ],
        out_shape=[
            jax.ShapeDtypeStruct((N, H), jnp.float32),
            jax.ShapeDtypeStruct((N, 1), jnp.float32),
        ],
    )(x, W, p0, p1)


def _tc_post(a0, a1, g, dis, b):
    N, H = g.shape
    nb = N // RB

    def body(a0_ref, a1_ref, g_ref, dis_ref, b_ref, t_ref, st_ref):
        i = pl.program_id(0)
        t = dis_ref[...] * (a0_ref[...] + a1_ref[...] + g_ref[...]) + b_ref[...][None, :]
        t_ref[...] = t

        @pl.when(i == 0)
        def _():
            st_ref[...] = jnp.zeros_like(st_ref)

        st_ref[0:1, :] += jnp.sum(t, axis=0, keepdims=True)
        st_ref[1:2, :] += jnp.sum(t * t, axis=0, keepdims=True)

    return pl.pallas_call(
        body,
        grid=(nb,),
        in_specs=[
            pl.BlockSpec((RB, H), lambda i: (i, 0)),
            pl.BlockSpec((RB, H), lambda i: (i, 0)),
            pl.BlockSpec((RB, H), lambda i: (i, 0)),
            pl.BlockSpec((RB, 1), lambda i: (i, 0)),
            pl.BlockSpec((H,), lambda i: (0,)),
        ],
        out_specs=[
            pl.BlockSpec((RB, H), lambda i: (i, 0)),
            pl.BlockSpec((2, H), lambda i: (0, 0)),
        ],
        out_shape=[
            jax.ShapeDtypeStruct((N, H), jnp.float32),
            jax.ShapeDtypeStruct((2, H), jnp.float32),
        ],
    )(a0, a1, g, dis, b)


def _tc_bn_mm(t, st, gamma, beta, dis, W):
    N, H = t.shape
    H2 = W.shape[1]
    nb = N // RB
    inv_n = 1.0 / N

    def body(t_ref, st_ref, ga_ref, be_ref, dis_ref, w_ref, g_ref):
        mu = st_ref[0:1, :] * inv_n
        var = st_ref[1:2, :] * inv_n - mu * mu
        hn = (t_ref[...] - mu) * lax.rsqrt(var + 1e-5) * ga_ref[...][None, :] \
            + be_ref[...][None, :]
        h = jnp.maximum(hn, 0.0)
        g_ref[...] = dis_ref[...] * jnp.dot(
            h, w_ref[...], preferred_element_type=jnp.float32)

    return pl.pallas_call(
        body,
        grid=(nb,),
        in_specs=[
            pl.BlockSpec((RB, H), lambda i: (i, 0)),
            pl.BlockSpec((2, H), lambda i: (0, 0)),
            pl.BlockSpec((H,), lambda i: (0,)),
            pl.BlockSpec((H,), lambda i: (0,)),
            pl.BlockSpec((RB, 1), lambda i: (i, 0)),
            pl.BlockSpec((H, H2), lambda i: (0, 0)),
        ],
        out_specs=pl.BlockSpec((RB, H2), lambda i: (i, 0)),
        out_shape=jax.ShapeDtypeStruct((N, H2), jnp.float32),
    )(t, st, gamma, beta, dis, W)


def _tc_final(t, st, gamma, beta, att_w, Wc, bc, batch_col, ng):
    N, H = t.shape
    nb = N // RB
    inv_n = 1.0 / N

    def body(t_ref, st_ref, ga_ref, be_ref, aw_ref, wc_ref, bc_ref, b_ref,
             att_ref, log_ref, pool_s):
        i = pl.program_id(0)
        mu = st_ref[0:1, :] * inv_n
        var = st_ref[1:2, :] * inv_n - mu * mu
        hn = (t_ref[...] - mu) * lax.rsqrt(var + 1e-5) * ga_ref[...][None, :] \
            + be_ref[...][None, :]
        h = jnp.maximum(hn, 0.0)
        att = jax.nn.sigmoid(
            jnp.dot(h, aw_ref[...], preferred_element_type=jnp.float32))
        att_ref[...] = att
        w = h * att
        oh = (lax.broadcasted_iota(jnp.int32, (RB, ng), 1)
              == b_ref[...]).astype(jnp.float32)
        part = lax.dot_general(oh, w, (((0,), (0,)), ((), ())),
                               preferred_element_type=jnp.float32,
                               precision=lax.Precision.HIGHEST)

        @pl.when(i == 0)
        def _():
            pool_s[...] = jnp.zeros_like(pool_s)

        pool_s[...] += part

        @pl.when(i == nb - 1)
        def _():
            log_ref[...] = jnp.dot(
                pool_s[...], wc_ref[...],
                preferred_element_type=jnp.float32) + bc_ref[...][None, :]

    return pl.pallas_call(
        body,
        grid=(nb,),
        in_specs=[
            pl.BlockSpec((RB, H), lambda i: (i, 0)),
            pl.BlockSpec((2, H), lambda i: (0, 0)),
            pl.BlockSpec((H,), lambda i: (0,)),
            pl.BlockSpec((H,), lambda i: (0,)),
            pl.BlockSpec((H, 1), lambda i: (0, 0)),
            pl.BlockSpec((H, 1), lambda i: (0, 0)),
            pl.BlockSpec((1,), lambda i: (0,)),
            pl.BlockSpec((RB, 1), lambda i: (i, 0)),
        ],
        out_specs=[
            pl.BlockSpec((RB, 1), lambda i: (i, 0)),
            pl.BlockSpec((ng, 1), lambda i: (0, 0)),
        ],
        out_shape=[
            jax.ShapeDtypeStruct((N, 1), jnp.float32),
            jax.ShapeDtypeStruct((ng, 1), jnp.float32),
        ],
        scratch_shapes=[pltpu.VMEM((ng, H), jnp.float32)],
    )(t, st, gamma, beta, att_w, Wc, bc, batch_col)


def kernel(x, edge_index, batch, W1, b1, W2, b2, bn1_gamma, bn1_beta,
           bn2_gamma, bn2_beta, att_w, Wc, bc):
    N, D = x.shape
    H = W1.shape[1]
    E = edge_index.shape[1]
    ng = 64
    batch_col = batch.reshape(N, 1)

    quantum = NC * NS * CH * NB
    e_tot = ((E + quantum - 1) // quantum) * quantum
    dst_p = jnp.concatenate(
        [edge_index[1],
         N + (jnp.arange(e_tot - E, dtype=jnp.int32) % 64)])

    n_pad = ((N + NS * 16 - 1) // (NS * 16)) * (NS * 16)
    zeros_nh = jnp.zeros((N, H), jnp.float32)

    e3 = jnp.stack([edge_index[0].reshape(-1, EC),
                    edge_index[1].reshape(-1, EC)], axis=1)
    degp = _sc_degree(dst_p, n_pad)
    p0 = degp[:N].reshape(N, 1)
    p1 = degp[n_pad:n_pad + N].reshape(N, 1)

    g1, dis = _tc_mm_scale(x, W1, p0, p1)
    acc1 = _sc_scatter(g1, e3, zeros_nh)
    t1, st1 = _tc_post(acc1[0], acc1[1], g1, dis, b1)
    g2 = _tc_bn_mm(t1, st1, bn1_gamma, bn1_beta, dis, W2)
    acc2 = _sc_scatter(g2, e3, zeros_nh)
    t2, st2 = _tc_post(acc2[0], acc2[1], g2, dis, b2)
    att, logits = _tc_final(t2, st2, bn2_gamma, bn2_beta, att_w, Wc,
                            bc, batch_col, ng)
    return (logits, att)
